# baseline (speedup 1.0000x reference)
import functools

import jax
import jax.numpy as jnp
from jax import lax
from jax.experimental import pallas as pl
from jax.experimental.pallas import tpu as pltpu
from jax.experimental.pallas import tpu_sc as plsc

_NC = 2
_NS = 16
_NW = _NC * _NS
_KC = 64
_L = 16

_GDN = lax.GatherDimensionNumbers(offset_dims=(), collapsed_slice_dims=(0,),
                                  start_index_map=(0,))


def _bcast_lane(vec, lane):
    idx = jnp.full((_L, 1), lane, jnp.int32)
    return lax.gather(vec, idx, _GDN, slice_sizes=(1,),
                      mode=lax.GatherScatterMode.PROMISE_IN_BOUNDS)


def _wcomb_body(w_ref, b_ref, o_ref):
    o_ref[...] = jnp.dot(w_ref[...], b_ref[...],
                         preferred_element_type=jnp.float32)


def _wcomb(wc, bases_flat, col_block):
    r, nb = wc.shape
    cols = bases_flat.shape[1]
    assert cols % col_block == 0
    return pl.pallas_call(
        _wcomb_body,
        grid=(cols // col_block,),
        in_specs=[
            pl.BlockSpec((r, nb), lambda i: (0, 0)),
            pl.BlockSpec((nb, col_block), lambda i: (0, i)),
        ],
        out_specs=pl.BlockSpec((r, col_block), lambda i: (0, i)),
        out_shape=jax.ShapeDtypeStruct((r, cols), jnp.float32),
    )(wc, bases_flat)


def _batmm_body(x_ref, w_ref, o_ref):
    o_ref[0] = jnp.dot(x_ref[...], w_ref[0],
                       preferred_element_type=jnp.float32)


def _batmm(x, w_all, bn):
    n, h = x.shape
    rr, _, ho = w_all.shape
    assert n % bn == 0
    return pl.pallas_call(
        _batmm_body,
        grid=(n // bn, rr),
        in_specs=[
            pl.BlockSpec((bn, h), lambda j, r: (j, 0)),
            pl.BlockSpec((1, h, ho), lambda j, r: (r, 0, 0)),
        ],
        out_specs=pl.BlockSpec((1, bn, ho), lambda j, r: (r, j, 0)),
        out_shape=jax.ShapeDtypeStruct((rr, n, ho), jnp.float32),
    )(x, w_all)


def _prep_body(s_ref, d_ref, r_ref, m_ref, idx_ref, dst_ref, nrm_ref,
               *, n, e, bkr):
    i = pl.program_id(0)
    pos = (i * bkr * 128
           + lax.broadcasted_iota(jnp.int32, (bkr, 128), 0) * 128
           + lax.broadcasted_iota(jnp.int32, (bkr, 128), 1))
    mask = pos < e
    idx_ref[...] = jnp.where(mask, r_ref[...] * n + s_ref[...], 0)
    dst_ref[...] = jnp.where(mask, d_ref[...], 0)
    nrm_ref[...] = jnp.where(mask, m_ref[...], 0.0)


def _prep_edges(src2d, dst2d, rel2d, nrm2d, n, e, e_pad):
    bkr = 16
    rows_in = src2d.shape[0]
    last_in = (rows_in + bkr - 1) // bkr - 1
    rows_out = e_pad // 128
    assert rows_out % bkr == 0
    in_spec = pl.BlockSpec((bkr, 128), lambda i: (jnp.minimum(i, last_in), 0))
    out_spec = pl.BlockSpec((bkr, 128), lambda i: (i, 0))
    shp = jax.ShapeDtypeStruct((rows_out, 128), jnp.int32)
    return pl.pallas_call(
        functools.partial(_prep_body, n=n, e=e, bkr=bkr),
        grid=(rows_out // bkr,),
        in_specs=[in_spec] * 4,
        out_specs=[out_spec] * 3,
        out_shape=[shp, shp, jax.ShapeDtypeStruct((rows_out, 128),
                                                  jnp.float32)],
    )(src2d, dst2d, rel2d, nrm2d)


def _epilogue_body(p_ref, sl_ref, b_ref, o_ref, *, relu):
    v = p_ref[0] + p_ref[1] + sl_ref[...] + b_ref[...]
    o_ref[...] = jnp.maximum(v, 0.0) if relu else v


def _epilogue(partials, selfloop, bias, relu, bn, sl_row=None):
    if sl_row is None:
        n, ho = selfloop.shape
        sl_spec = pl.BlockSpec((bn, ho), lambda i: (i, 0))
        sl_slice = lambda ref: ref[...]
    else:
        _, n, ho = selfloop.shape
        sl_spec = pl.BlockSpec((1, bn, ho), lambda i: (sl_row, i, 0))
        sl_slice = lambda ref: ref[0]
    assert n % bn == 0

    def body(p_ref, sl_ref, b_ref, o_ref):
        v = p_ref[0] + p_ref[1] + sl_slice(sl_ref) + b_ref[...]
        o_ref[...] = jnp.maximum(v, 0.0) if relu else v

    return pl.pallas_call(
        body,
        grid=(n // bn,),
        in_specs=[
            pl.BlockSpec((2, bn, ho), lambda i: (0, i, 0)),
            sl_spec,
            pl.BlockSpec((1, ho), lambda i: (0, 0)),
        ],
        out_specs=pl.BlockSpec((bn, ho), lambda i: (i, 0)),
        out_shape=jax.ShapeDtypeStruct((n, ho), jnp.float32),
    )(partials, selfloop, bias)


def _sc_body(n, n_pad, nchunk, hout, table, idx_h, dst_h, norm_h,
             out_h, idx_v, dst_v, norm_v, rows_v, acc,
             gsem0, gsem1, ssem0, ssem1):
    c = lax.axis_index("c")
    s = lax.axis_index("s")
    wid = c * _NS + s
    ep = nchunk * _KC
    base = wid * ep
    nrows = n_pad // _NS
    row0 = s * nrows
    q8 = hout // _L
    gsems = (gsem0, gsem1)
    ssems = (ssem0, ssem1)

    pltpu.sync_copy(idx_h.at[pl.ds(base, ep)], idx_v)
    pltpu.sync_copy(norm_h.at[pl.ds(base, ep)], norm_v)
    pltpu.sync_copy(dst_h.at[wid], dst_v)

    def start_gather(i, b):
        pltpu.async_copy(table.at[idx_v.at[pl.ds(i * _KC, _KC)]],
                         rows_v.at[b], gsems[b])

    def wait_gather(i, b):
        pltpu.make_async_copy(table.at[idx_v.at[pl.ds(i * _KC, _KC)]],
                              rows_v.at[b], gsems[b]).wait()

    def start_scatter(i, b):
        pltpu.async_copy(rows_v.at[b], acc.at[dst_v.at[i]], ssems[b],
                         add=True)

    def wait_scatter(i, b):
        pltpu.make_async_copy(rows_v.at[b], acc.at[dst_v.at[i]],
                              ssems[b]).wait()

    def scale(i, b):
        def sbody(jj, _):
            nv = norm_v[pl.ds(i * _KC + jj * _L, _L)]
            for l in range(_L):
                scal = _bcast_lane(nv, l)
                row = jj * _L + l
                for q in range(q8):
                    cur = rows_v[b, row, pl.ds(q * _L, _L)]
                    rows_v[b, row, pl.ds(q * _L, _L)] = cur * scal
            return 0
        lax.fori_loop(0, _KC // _L, sbody, 0)

    def zbody(j, _):
        for q in range(q8):
            rows_v[0, j, pl.ds(q * _L, _L)] = jnp.zeros((_L,), jnp.float32)
        return 0
    lax.fori_loop(0, _KC, zbody, 0)
    off = 0
    while off < nrows:
        sz = min(_KC, nrows - off)
        pltpu.sync_copy(rows_v.at[0].at[pl.ds(0, sz)],
                        acc.at[pl.ds(row0 + off, sz)])
        off += sz
    plsc.subcore_barrier()

    def pair(gi, _):
        i0 = gi * 2
        wait_gather(i0, 0)

        @pl.when(gi > 0)
        def _():
            wait_scatter(i0 - 1, 1)
        start_gather(i0 + 1, 1)
        scale(i0, 0)
        start_scatter(i0, 0)

        wait_gather(i0 + 1, 1)
        wait_scatter(i0, 0)

        @pl.when(gi < nchunk // 2 - 1)
        def _():
            start_gather(i0 + 2, 0)
        scale(i0 + 1, 1)
        start_scatter(i0 + 1, 1)
        return 0

    start_gather(0, 0)
    lax.fori_loop(0, nchunk // 2, pair, 0)
    wait_scatter(nchunk - 1, 1)
    plsc.subcore_barrier()

    off = 0
    while off < nrows:
        sz = min(_KC, nrows - off)
        pltpu.sync_copy(acc.at[pl.ds(row0 + off, sz)],
                        out_h.at[c, pl.ds(row0 + off, sz)])
        off += sz


def _sc_gather_segsum(table, idx, dst3d, norm, n, hout):
    e_pad = idx.shape[0]
    nchunk = e_pad // (_NW * _KC)
    ep = nchunk * _KC
    n_pad = ((n + _NS * 8 - 1) // (_NS * 8)) * (_NS * 8)
    mesh = plsc.VectorSubcoreMesh(core_axis_name="c", subcore_axis_name="s")
    body = functools.partial(_sc_body, n, n_pad, nchunk, hout)
    return pl.kernel(
        body,
        out_type=jax.ShapeDtypeStruct((_NC, n_pad, hout), jnp.float32),
        mesh=mesh,
        compiler_params=pltpu.CompilerParams(use_tc_tiling_on_sc=False),
        scratch_types=[
            pltpu.VMEM((ep,), jnp.int32),
            pltpu.VMEM((nchunk, _KC), jnp.int32),
            pltpu.VMEM((ep,), jnp.float32),
            pltpu.VMEM((2, _KC, hout), jnp.float32),
            pltpu.VMEM_SHARED((n_pad, hout), jnp.float32),
            pltpu.SemaphoreType.DMA,
            pltpu.SemaphoreType.DMA,
            pltpu.SemaphoreType.DMA,
            pltpu.SemaphoreType.DMA,
        ],
    )(table, idx, dst3d, norm)


def kernel(g, h, r, norm, w_comp0, bases0, loop0, bias0,
           w_comp1, bases1, loop1, bias1,
           w_comp2, bases2, loop2, bias2):
    nb, n, hh = bases0.shape
    rr = w_comp0.shape[0]
    c = bases2.shape[2]
    e = g.shape[1]

    quant = _NW * _KC * 2
    e_pad = ((e + quant - 1) // quant) * quant
    idx2d, dst2d, nrm2d = _prep_edges(
        g[0].astype(jnp.int32).reshape(-1, 128),
        g[1].astype(jnp.int32).reshape(-1, 128),
        r.astype(jnp.int32).reshape(-1, 128),
        norm.astype(jnp.float32).reshape(-1, 128),
        n, e, e_pad)
    idx = idx2d.reshape(-1)
    nrm = nrm2d.reshape(-1)
    dst3d = dst2d.reshape(_NW, e_pad // (_NW * _KC), _KC)

    table0 = _wcomb(w_comp0, bases0.reshape(nb, n * hh), col_block=10240)
    p0 = _sc_gather_segsum(table0.reshape(rr * n, hh), idx, dst3d, nrm, n, hh)
    x0 = _epilogue(p0, loop0, bias0.reshape(1, hh), relu=True, bn=1000)

    w1 = _wcomb(w_comp1, bases1.reshape(nb, hh * hh), col_block=hh * hh)
    w1_all = jnp.concatenate([w1.reshape(rr, hh, hh), loop1[None]], axis=0)
    t1 = _batmm(x0, w1_all, bn=1000)
    p1 = _sc_gather_segsum(t1.reshape((rr + 1) * n, hh), idx, dst3d, nrm, n, hh)
    x1 = _epilogue(p1, t1, bias1.reshape(1, hh), relu=True, bn=1000, sl_row=rr)

    w2 = _wcomb(w_comp2, bases2.reshape(nb, hh * c), col_block=hh * c)
    w2_all = jnp.concatenate([w2.reshape(rr, hh, c), loop2[None]], axis=0)
    t2 = _batmm(x1, w2_all, bn=1000)
    p2 = _sc_gather_segsum(t2.reshape((rr + 1) * n, c), idx, dst3d, nrm, n, c)
    return _epilogue(p2, t2, bias2.reshape(1, c), relu=False, bn=1000, sl_row=rr)


# speedup vs baseline: 1.0847x; 1.0847x over previous
import functools

import jax
import jax.numpy as jnp
from jax import lax
from jax.experimental import pallas as pl
from jax.experimental.pallas import tpu as pltpu
from jax.experimental.pallas import tpu_sc as plsc

_NC = 2
_NS = 16
_NW = _NC * _NS
_KC = 64
_L = 16

_GDN = lax.GatherDimensionNumbers(offset_dims=(), collapsed_slice_dims=(0,),
                                  start_index_map=(0,))


def _bcast_lane(vec, lane):
    idx = jnp.full((_L, 1), lane, jnp.int32)
    return lax.gather(vec, idx, _GDN, slice_sizes=(1,),
                      mode=lax.GatherScatterMode.PROMISE_IN_BOUNDS)


def _btab_body(wc_ref, b_ref, o_ref, *, nb):
    r = pl.program_id(1)
    acc = wc_ref[r, 0] * b_ref[0]
    for b in range(1, nb):
        acc = acc + wc_ref[r, b] * b_ref[b]
    o_ref[0] = acc


def _btab(wc, bases, bn):
    rr, nb = wc.shape
    _, m, h = bases.shape
    assert m % bn == 0
    return pl.pallas_call(
        functools.partial(_btab_body, nb=nb),
        grid=(m // bn, rr),
        in_specs=[
            pl.BlockSpec(memory_space=pltpu.SMEM),
            pl.BlockSpec((nb, bn, h), lambda j, r: (0, j, 0)),
        ],
        out_specs=pl.BlockSpec((1, bn, h), lambda j, r: (r, j, 0)),
        out_shape=jax.ShapeDtypeStruct((rr, m, h), jnp.float32),
    )(wc, bases)


def _batmm_body(x_ref, w_ref, o_ref):
    o_ref[0] = jnp.dot(x_ref[...], w_ref[0],
                       preferred_element_type=jnp.float32)


def _batmm(x, w_all, bn):
    n, h = x.shape
    rr, _, ho = w_all.shape
    assert n % bn == 0
    return pl.pallas_call(
        _batmm_body,
        grid=(n // bn, rr),
        in_specs=[
            pl.BlockSpec((bn, h), lambda j, r: (j, 0)),
            pl.BlockSpec((1, h, ho), lambda j, r: (r, 0, 0)),
        ],
        out_specs=pl.BlockSpec((1, bn, ho), lambda j, r: (r, j, 0)),
        out_shape=jax.ShapeDtypeStruct((rr, n, ho), jnp.float32),
    )(x, w_all)


def _idx_body(s_ref, r_ref, o_ref, *, n):
    o_ref[...] = r_ref[...] * n + s_ref[...]


def _flat_idx(src2d, rel2d, n):
    return pl.pallas_call(
        functools.partial(_idx_body, n=n),
        out_shape=jax.ShapeDtypeStruct(src2d.shape, jnp.int32),
    )(src2d, rel2d)


def _epilogue(partials, selfloop, bias, relu, bn, sl_row=None):
    if sl_row is None:
        n, ho = selfloop.shape
        sl_spec = pl.BlockSpec((bn, ho), lambda i: (i, 0))
        sl_slice = lambda ref: ref[...]
    else:
        _, n, ho = selfloop.shape
        sl_spec = pl.BlockSpec((1, bn, ho), lambda i: (sl_row, i, 0))
        sl_slice = lambda ref: ref[0]
    assert n % bn == 0

    def body(p_ref, sl_ref, b_ref, o_ref):
        v = p_ref[0] + p_ref[1] + sl_slice(sl_ref) + b_ref[...]
        o_ref[...] = jnp.maximum(v, 0.0) if relu else v

    return pl.pallas_call(
        body,
        grid=(n // bn,),
        in_specs=[
            pl.BlockSpec((2, bn, ho), lambda i: (0, i, 0)),
            sl_spec,
            pl.BlockSpec((1, ho), lambda i: (0, 0)),
        ],
        out_specs=pl.BlockSpec((bn, ho), lambda i: (i, 0)),
        out_shape=jax.ShapeDtypeStruct((n, ho), jnp.float32),
    )(partials, selfloop, bias)


def _sc_body(n, n_pad, nchunk, hout, table, idx_h, dst_h, norm_h,
             out_h, idx_v, dst_v, norm_v, rows_v, acc,
             gsem0, gsem1, ssem0, ssem1):
    c = lax.axis_index("c")
    s = lax.axis_index("s")
    wid = c * _NS + s
    ep = nchunk * _KC
    base = wid * ep
    nrows = n_pad // _NS
    row0 = s * nrows
    q8 = hout // _L
    gsems = (gsem0, gsem1)
    ssems = (ssem0, ssem1)

    pltpu.sync_copy(idx_h.at[pl.ds(base, ep)], idx_v)
    pltpu.sync_copy(norm_h.at[pl.ds(base, ep)], norm_v)
    pltpu.sync_copy(dst_h.at[wid], dst_v)

    def start_gather(i, b):
        pltpu.async_copy(table.at[idx_v.at[pl.ds(i * _KC, _KC)]],
                         rows_v.at[b], gsems[b])

    def wait_gather(i, b):
        pltpu.make_async_copy(table.at[idx_v.at[pl.ds(i * _KC, _KC)]],
                              rows_v.at[b], gsems[b]).wait()

    def start_scatter(i, b):
        pltpu.async_copy(rows_v.at[b], acc.at[dst_v.at[i]], ssems[b],
                         add=True)

    def wait_scatter(i, b):
        pltpu.make_async_copy(rows_v.at[b], acc.at[dst_v.at[i]],
                              ssems[b]).wait()

    def scale(i, b):
        def sbody(jj, _):
            nv = norm_v[pl.ds(i * _KC + jj * _L, _L)]
            for l in range(_L):
                scal = _bcast_lane(nv, l)
                row = jj * _L + l
                for q in range(q8):
                    cur = rows_v[b, row, pl.ds(q * _L, _L)]
                    rows_v[b, row, pl.ds(q * _L, _L)] = cur * scal
            return 0
        lax.fori_loop(0, _KC // _L, sbody, 0)

    def zbody(j, _):
        for q in range(q8):
            rows_v[0, j, pl.ds(q * _L, _L)] = jnp.zeros((_L,), jnp.float32)
        return 0
    lax.fori_loop(0, _KC, zbody, 0)
    off = 0
    while off < nrows:
        sz = min(_KC, nrows - off)
        pltpu.sync_copy(rows_v.at[0].at[pl.ds(0, sz)],
                        acc.at[pl.ds(row0 + off, sz)])
        off += sz
    plsc.subcore_barrier()

    def pair(gi, _):
        i0 = gi * 2
        wait_gather(i0, 0)

        @pl.when(gi > 0)
        def _():
            wait_scatter(i0 - 1, 1)
        start_gather(i0 + 1, 1)
        scale(i0, 0)
        start_scatter(i0, 0)

        wait_gather(i0 + 1, 1)
        wait_scatter(i0, 0)

        @pl.when(gi < nchunk // 2 - 1)
        def _():
            start_gather(i0 + 2, 0)
        scale(i0 + 1, 1)
        start_scatter(i0 + 1, 1)
        return 0

    start_gather(0, 0)
    lax.fori_loop(0, nchunk // 2, pair, 0)
    wait_scatter(nchunk - 1, 1)
    plsc.subcore_barrier()

    off = 0
    while off < nrows:
        sz = min(_KC, nrows - off)
        pltpu.sync_copy(acc.at[pl.ds(row0 + off, sz)],
                        out_h.at[c, pl.ds(row0 + off, sz)])
        off += sz


def _sc_gather_segsum(table, idx, dst3d, norm, n, hout):
    e_pad = idx.shape[0]
    nchunk = e_pad // (_NW * _KC)
    ep = nchunk * _KC
    n_pad = ((n + _NS * 8 - 1) // (_NS * 8)) * (_NS * 8)
    mesh = plsc.VectorSubcoreMesh(core_axis_name="c", subcore_axis_name="s")
    body = functools.partial(_sc_body, n, n_pad, nchunk, hout)
    return pl.kernel(
        body,
        out_type=jax.ShapeDtypeStruct((_NC, n_pad, hout), jnp.float32),
        mesh=mesh,
        compiler_params=pltpu.CompilerParams(use_tc_tiling_on_sc=False),
        scratch_types=[
            pltpu.VMEM((ep,), jnp.int32),
            pltpu.VMEM((nchunk, _KC), jnp.int32),
            pltpu.VMEM((ep,), jnp.float32),
            pltpu.VMEM((2, _KC, hout), jnp.float32),
            pltpu.VMEM_SHARED((n_pad, hout), jnp.float32),
            pltpu.SemaphoreType.DMA,
            pltpu.SemaphoreType.DMA,
            pltpu.SemaphoreType.DMA,
            pltpu.SemaphoreType.DMA,
        ],
    )(table, idx, dst3d, norm)


def kernel(g, h, r, norm, w_comp0, bases0, loop0, bias0,
           w_comp1, bases1, loop1, bias1,
           w_comp2, bases2, loop2, bias2):
    nb, n, hh = bases0.shape
    rr = w_comp0.shape[0]
    c = bases2.shape[2]
    e = g.shape[1]

    quant = _NW * _KC * 2
    e_pad = ((e + quant - 1) // quant) * quant
    pad = e_pad - e
    srcp = jnp.pad(g[0].astype(jnp.int32), (0, pad))
    dst = jnp.pad(g[1].astype(jnp.int32), (0, pad))
    rel = jnp.pad(r.astype(jnp.int32), (0, pad))
    nrm = jnp.pad(norm.reshape(-1).astype(jnp.float32), (0, pad))
    dst3d = dst.reshape(_NW, e_pad // (_NW * _KC), _KC)
    idx = _flat_idx(srcp.reshape(-1, 128), rel.reshape(-1, 128), n).reshape(-1)

    table0 = _btab(w_comp0, bases0, bn=1000)
    p0 = _sc_gather_segsum(table0.reshape(rr * n, hh), idx, dst3d, nrm, n, hh)
    x0 = _epilogue(p0, loop0, bias0.reshape(1, hh), relu=True, bn=1000)

    w1 = _btab(w_comp1, bases1, bn=hh)
    w1_all = jnp.concatenate([w1, loop1[None]], axis=0)
    t1 = _batmm(x0, w1_all, bn=1000)
    p1 = _sc_gather_segsum(t1.reshape((rr + 1) * n, hh), idx, dst3d, nrm, n, hh)
    x1 = _epilogue(p1, t1, bias1.reshape(1, hh), relu=True, bn=1000, sl_row=rr)

    w2 = _btab(w_comp2, bases2, bn=hh)
    w2_all = jnp.concatenate([w2, loop2[None]], axis=0)
    t2 = _batmm(x1, w2_all, bn=1000)
    p2 = _sc_gather_segsum(t2.reshape((rr + 1) * n, c), idx, dst3d, nrm, n, c)
    return _epilogue(p2, t2, bias2.reshape(1, c), relu=False, bn=1000, sl_row=rr)


# speedup vs baseline: 1.2251x; 1.1295x over previous
import functools

import jax
import jax.numpy as jnp
from jax import lax
from jax.experimental import pallas as pl
from jax.experimental.pallas import tpu as pltpu
from jax.experimental.pallas import tpu_sc as plsc

_NC = 2
_NS = 16
_NW = _NC * _NS
_KC = 64
_L = 16

_GDN = lax.GatherDimensionNumbers(offset_dims=(), collapsed_slice_dims=(0,),
                                  start_index_map=(0,))


def _bcast_lane(vec, lane):
    idx = jnp.full((_L, 1), lane, jnp.int32)
    return lax.gather(vec, idx, _GDN, slice_sizes=(1,),
                      mode=lax.GatherScatterMode.PROMISE_IN_BOUNDS)


def _btab_body(wc_ref, b_ref, o_ref, *, nb):
    r = pl.program_id(1)
    acc = wc_ref[r, 0] * b_ref[0]
    for b in range(1, nb):
        acc = acc + wc_ref[r, b] * b_ref[b]
    o_ref[0] = acc


def _btab(wc, bases, bn):
    rr, nb = wc.shape
    _, m, h = bases.shape
    assert m % bn == 0
    return pl.pallas_call(
        functools.partial(_btab_body, nb=nb),
        grid=(m // bn, rr),
        in_specs=[
            pl.BlockSpec(memory_space=pltpu.SMEM),
            pl.BlockSpec((nb, bn, h), lambda j, r: (0, j, 0)),
        ],
        out_specs=pl.BlockSpec((1, bn, h), lambda j, r: (r, j, 0)),
        out_shape=jax.ShapeDtypeStruct((rr, m, h), jnp.float32),
    )(wc, bases)


def _batmm_body(x_ref, w_ref, o_ref):
    o_ref[0] = jnp.dot(x_ref[...], w_ref[0],
                       preferred_element_type=jnp.float32)


def _ep_batmm(partials, selfloop, bias, w_all, bn, sl_row=None):
    rra, h, ho = w_all.shape
    if sl_row is None:
        n = selfloop.shape[0]
        sl_spec = pl.BlockSpec((bn, h), lambda j, r: (j, 0))
        sl_slice = lambda ref: ref[...]
    else:
        n = selfloop.shape[1]
        sl_spec = pl.BlockSpec((1, bn, h), lambda j, r: (sl_row, j, 0))
        sl_slice = lambda ref: ref[0]
    assert n % bn == 0

    def body(p_ref, sl_ref, b_ref, w_ref, o_ref, x_ref):
        @pl.when(pl.program_id(1) == 0)
        def _():
            v = p_ref[0] + p_ref[1] + sl_slice(sl_ref) + b_ref[...]
            x_ref[...] = jnp.maximum(v, 0.0)
        o_ref[0] = jnp.dot(x_ref[...], w_ref[0],
                           preferred_element_type=jnp.float32)

    return pl.pallas_call(
        body,
        grid=(n // bn, rra),
        in_specs=[
            pl.BlockSpec((2, bn, h), lambda j, r: (0, j, 0)),
            sl_spec,
            pl.BlockSpec((1, h), lambda j, r: (0, 0)),
            pl.BlockSpec((1, h, ho), lambda j, r: (r, 0, 0)),
        ],
        out_specs=pl.BlockSpec((1, bn, ho), lambda j, r: (r, j, 0)),
        out_shape=jax.ShapeDtypeStruct((rra, n, ho), jnp.float32),
        scratch_shapes=[pltpu.VMEM((bn, h), jnp.float32)],
    )(partials, selfloop, bias, w_all)


def _idx_body(s_ref, r_ref, o_ref, *, n):
    o_ref[...] = r_ref[...] * n + s_ref[...]


def _flat_idx(src2d, rel2d, n):
    return pl.pallas_call(
        functools.partial(_idx_body, n=n),
        out_shape=jax.ShapeDtypeStruct(src2d.shape, jnp.int32),
    )(src2d, rel2d)


def _epilogue(partials, selfloop, bias, relu, bn, sl_row=None):
    if sl_row is None:
        n, ho = selfloop.shape
        sl_spec = pl.BlockSpec((bn, ho), lambda i: (i, 0))
        sl_slice = lambda ref: ref[...]
    else:
        _, n, ho = selfloop.shape
        sl_spec = pl.BlockSpec((1, bn, ho), lambda i: (sl_row, i, 0))
        sl_slice = lambda ref: ref[0]
    assert n % bn == 0

    def body(p_ref, sl_ref, b_ref, o_ref):
        v = p_ref[0] + p_ref[1] + sl_slice(sl_ref) + b_ref[...]
        o_ref[...] = jnp.maximum(v, 0.0) if relu else v

    return pl.pallas_call(
        body,
        grid=(n // bn,),
        in_specs=[
            pl.BlockSpec((2, bn, ho), lambda i: (0, i, 0)),
            sl_spec,
            pl.BlockSpec((1, ho), lambda i: (0, 0)),
        ],
        out_specs=pl.BlockSpec((bn, ho), lambda i: (i, 0)),
        out_shape=jax.ShapeDtypeStruct((n, ho), jnp.float32),
    )(partials, selfloop, bias)


def _sc_body(n, n_pad, nchunk, hout, kc, table, idx_h, dst_h, norm_h,
             out_h, idx_v, dst_v, norm_v, rows_v, acc,
             gsem0, gsem1, ssem0, ssem1):
    c = lax.axis_index("c")
    s = lax.axis_index("s")
    wid = c * _NS + s
    ep = nchunk * kc
    base = wid * ep
    nrows = n_pad // _NS
    row0 = s * nrows
    q8 = hout // _L
    gsems = (gsem0, gsem1)
    ssems = (ssem0, ssem1)

    pltpu.sync_copy(idx_h.at[pl.ds(base, ep)], idx_v)
    pltpu.sync_copy(norm_h.at[pl.ds(base, ep)], norm_v)
    pltpu.sync_copy(dst_h.at[wid], dst_v)

    def start_gather(i, b):
        pltpu.async_copy(table.at[idx_v.at[pl.ds(i * kc, kc)]],
                         rows_v.at[b], gsems[b])

    def wait_gather(i, b):
        pltpu.make_async_copy(table.at[idx_v.at[pl.ds(i * kc, kc)]],
                              rows_v.at[b], gsems[b]).wait()

    def start_scatter(i, b):
        pltpu.async_copy(rows_v.at[b], acc.at[dst_v.at[i]], ssems[b],
                         add=True)

    def wait_scatter(i, b):
        pltpu.make_async_copy(rows_v.at[b], acc.at[dst_v.at[i]],
                              ssems[b]).wait()

    def scale(i, b):
        def sbody(jj, _):
            nv = norm_v[pl.ds(i * kc + jj * _L, _L)]
            for l in range(_L):
                scal = _bcast_lane(nv, l)
                row = jj * _L + l
                for q in range(q8):
                    cur = rows_v[b, row, pl.ds(q * _L, _L)]
                    rows_v[b, row, pl.ds(q * _L, _L)] = cur * scal
            return 0
        lax.fori_loop(0, kc // _L, sbody, 0)

    def zbody(j, _):
        for q in range(q8):
            rows_v[0, j, pl.ds(q * _L, _L)] = jnp.zeros((_L,), jnp.float32)
        return 0
    lax.fori_loop(0, kc, zbody, 0)
    off = 0
    while off < nrows:
        sz = min(kc, nrows - off)
        pltpu.sync_copy(rows_v.at[0].at[pl.ds(0, sz)],
                        acc.at[pl.ds(row0 + off, sz)])
        off += sz
    plsc.subcore_barrier()

    def pair(gi, _):
        i0 = gi * 2
        wait_gather(i0, 0)

        @pl.when(gi > 0)
        def _():
            wait_scatter(i0 - 1, 1)
        start_gather(i0 + 1, 1)
        scale(i0, 0)
        start_scatter(i0, 0)

        wait_gather(i0 + 1, 1)
        wait_scatter(i0, 0)

        @pl.when(i0 + 2 < nchunk)
        def _():
            start_gather(i0 + 2, 0)
        scale(i0 + 1, 1)
        start_scatter(i0 + 1, 1)
        return 0

    start_gather(0, 0)
    lax.fori_loop(0, nchunk // 2, pair, 0)
    if nchunk % 2:
        wait_gather(nchunk - 1, 0)
        scale(nchunk - 1, 0)
        start_scatter(nchunk - 1, 0)
        wait_scatter(nchunk - 1, 0)
    wait_scatter(2 * (nchunk // 2) - 1, 1)
    plsc.subcore_barrier()

    off = 0
    while off < nrows:
        sz = min(kc, nrows - off)
        pltpu.sync_copy(acc.at[pl.ds(row0 + off, sz)],
                        out_h.at[c, pl.ds(row0 + off, sz)])
        off += sz


def _sc_gather_segsum(table, idx, dst3d, norm, n, hout, kc=_KC):
    e_pad = idx.shape[0]
    nchunk = e_pad // (_NW * kc)
    ep = nchunk * kc
    n_pad = ((n + _NS * 8 - 1) // (_NS * 8)) * (_NS * 8)
    mesh = plsc.VectorSubcoreMesh(core_axis_name="c", subcore_axis_name="s")
    body = functools.partial(_sc_body, n, n_pad, nchunk, hout, kc)
    return pl.kernel(
        body,
        out_type=jax.ShapeDtypeStruct((_NC, n_pad, hout), jnp.float32),
        mesh=mesh,
        compiler_params=pltpu.CompilerParams(use_tc_tiling_on_sc=False),
        scratch_types=[
            pltpu.VMEM((ep,), jnp.int32),
            pltpu.VMEM((nchunk, kc), jnp.int32),
            pltpu.VMEM((ep,), jnp.float32),
            pltpu.VMEM((2, kc, hout), jnp.float32),
            pltpu.VMEM_SHARED((n_pad, hout), jnp.float32),
            pltpu.SemaphoreType.DMA,
            pltpu.SemaphoreType.DMA,
            pltpu.SemaphoreType.DMA,
            pltpu.SemaphoreType.DMA,
        ],
    )(table, idx, dst3d, norm)


def kernel(g, h, r, norm, w_comp0, bases0, loop0, bias0,
           w_comp1, bases1, loop1, bias1,
           w_comp2, bases2, loop2, bias2):
    nb, n, hh = bases0.shape
    rr = w_comp0.shape[0]
    c = bases2.shape[2]
    e = g.shape[1]

    quant = _NW * _KC * 2
    e_pad = ((e + quant - 1) // quant) * quant
    pad = e_pad - e
    srcp = jnp.pad(g[0].astype(jnp.int32), (0, pad))
    dst = jnp.pad(g[1].astype(jnp.int32), (0, pad))
    rel = jnp.pad(r.astype(jnp.int32), (0, pad))
    nrm = jnp.pad(norm.reshape(-1).astype(jnp.float32), (0, pad))
    dst3d = dst.reshape(_NW, e_pad // (_NW * _KC), _KC)
    idx = _flat_idx(srcp.reshape(-1, 128), rel.reshape(-1, 128), n).reshape(-1)

    table0 = _btab(w_comp0, bases0, bn=1000)
    p0 = _sc_gather_segsum(table0.reshape(rr * n, hh), idx, dst3d, nrm, n, hh)

    w1 = _btab(w_comp1, bases1, bn=hh)
    w1_all = jnp.concatenate([w1, loop1[None]], axis=0)
    t1 = _ep_batmm(p0, loop0, bias0.reshape(1, hh), w1_all, bn=1000)
    p1 = _sc_gather_segsum(t1.reshape((rr + 1) * n, hh), idx, dst3d, nrm, n, hh)

    w2 = _btab(w_comp2, bases2, bn=hh)
    w2_all = jnp.concatenate([w2, loop2[None]], axis=0)
    t2 = _ep_batmm(p1, t1, bias1.reshape(1, hh), w2_all, bn=1000, sl_row=rr)
    dst3d_w = dst.reshape(_NW, e_pad // (_NW * 128), 128)
    p2 = _sc_gather_segsum(t2.reshape((rr + 1) * n, c), idx, dst3d_w, nrm,
                           n, c, kc=128)
    return _epilogue(p2, t2, bias2.reshape(1, c), relu=False, bn=1000, sl_row=rr)
